# baseline (speedup 1.0000x reference)
import functools

import jax
import jax.numpy as jnp
from jax.experimental import pallas as pl
from jax.experimental.pallas import tpu as pltpu


def _routing_body(x_ref, Wgr_ref, bgr_ref, cw_ref, *, E):
    xf = x_ref[...]
    logits = (
        jnp.dot(xf, Wgr_ref[...], preferred_element_type=jnp.float32)
        + bgr_ref[...]
    )
    gate = logits[:, :E]
    route = logits[:, E : E + 2]

    gmax = jnp.max(gate, axis=-1, keepdims=True)
    gexp = jnp.exp(gate - gmax)
    probs = gexp / jnp.sum(gexp, axis=-1, keepdims=True)

    eidx = jax.lax.broadcasted_iota(jnp.int32, probs.shape, 1)
    m1 = jnp.max(probs, axis=-1, keepdims=True)
    i1 = jnp.min(jnp.where(probs == m1, eidx, E), axis=-1, keepdims=True)
    mask1 = eidx == i1
    rest = jnp.where(mask1, -jnp.inf, probs)
    m2 = jnp.max(rest, axis=-1, keepdims=True)
    i2 = jnp.min(jnp.where(rest == m2, eidx, E), axis=-1, keepdims=True)
    mask2 = eidx == i2

    denom = m1 + m2
    wpe = (
        jnp.where(mask1, m1, 0.0) + jnp.where(mask2, m2, 0.0)
    ) / denom

    rmax = jnp.max(route, axis=-1, keepdims=True)
    rexp = jnp.exp(route - rmax)
    rp = rexp / jnp.sum(rexp, axis=-1, keepdims=True)
    moe_w = rp[:, 0:1]
    dense_w = rp[:, 1:2]

    cw_ref[:, :E] = moe_w * wpe
    cw_ref[:, E : E + 1] = dense_w


def _expert_body(x_ref, W1_ref, b1_ref, W2_ref, b2_ref, cw_ref, out_ref, *, BT, NE):
    e = pl.program_id(0)
    t = pl.program_id(1)
    rows = pl.ds(t * BT, BT)

    x = x_ref[rows, :]
    h = jnp.maximum(
        jnp.dot(x, W1_ref[0], preferred_element_type=jnp.float32)
        + b1_ref[0, 0, :],
        0.0,
    )
    o = (
        jnp.dot(h, W2_ref[0], preferred_element_type=jnp.float32)
        + b2_ref[0, 0, :]
    )

    cw = cw_ref[rows, :]
    lane = jax.lax.broadcasted_iota(jnp.int32, cw.shape, 1)
    w = jnp.sum(jnp.where(lane == e, cw, 0.0), axis=-1, keepdims=True)
    contrib = w * o

    @pl.when(e == 0)
    def _():
        out_ref[rows, :] = contrib

    @pl.when(e > 0)
    def _():
        out_ref[rows, :] += contrib


def kernel(x, Wg, bg, W1, b1, W2, b2, Wd1, bd1, Wd2, bd2, Wr, br):
    B_, S_, D_ = x.shape
    T = B_ * S_
    E = W1.shape[0]
    FF = W1.shape[2]
    NE = E + 1
    xf = x.reshape(T, D_)

    Wgr = jnp.concatenate([Wg, Wr], axis=1)
    bgr = jnp.concatenate([bg, br], axis=0)

    cw = pl.pallas_call(
        functools.partial(_routing_body, E=E),
        out_shape=jax.ShapeDtypeStruct((T, NE), jnp.float32),
    )(xf, Wgr, bgr[None, :])

    W1s = jnp.concatenate([W1, Wd1[None]], axis=0)
    b1s = jnp.concatenate([b1, bd1[None]], axis=0).reshape(NE, 1, FF)
    W2s = jnp.concatenate([W2, Wd2[None]], axis=0)
    b2s = jnp.concatenate([b2, bd2[None]], axis=0).reshape(NE, 1, D_)

    BT = 256
    TB = T // BT

    out = pl.pallas_call(
        functools.partial(_expert_body, BT=BT, NE=NE),
        grid=(NE, TB),
        in_specs=[
            pl.BlockSpec((T, D_), lambda e, t: (0, 0)),
            pl.BlockSpec((1, D_, FF), lambda e, t: (e, 0, 0)),
            pl.BlockSpec((1, 1, FF), lambda e, t: (e, 0, 0)),
            pl.BlockSpec((1, FF, D_), lambda e, t: (e, 0, 0)),
            pl.BlockSpec((1, 1, D_), lambda e, t: (e, 0, 0)),
            pl.BlockSpec((T, NE), lambda e, t: (0, 0)),
        ],
        out_specs=pl.BlockSpec((T, D_), lambda e, t: (0, 0)),
        out_shape=jax.ShapeDtypeStruct((T, D_), jnp.float32),
    )(xf, W1s, b1s, W2s, b2s, cw)

    return out.reshape(B_, S_, D_)


# speedup vs baseline: 1.0038x; 1.0038x over previous
import functools

import jax
import jax.numpy as jnp
from jax.experimental import pallas as pl
from jax.experimental.pallas import tpu as pltpu


def _routing_body(x_ref, Wgr_ref, bgr_ref, cw_ref, *, E):
    xf = x_ref[...]
    logits = (
        jnp.dot(xf, Wgr_ref[...], preferred_element_type=jnp.float32)
        + bgr_ref[...]
    )
    gate = logits[:, :E]
    route = logits[:, E : E + 2]

    gmax = jnp.max(gate, axis=-1, keepdims=True)
    gexp = jnp.exp(gate - gmax)
    probs = gexp / jnp.sum(gexp, axis=-1, keepdims=True)

    eidx = jax.lax.broadcasted_iota(jnp.int32, probs.shape, 1)
    m1 = jnp.max(probs, axis=-1, keepdims=True)
    i1 = jnp.min(jnp.where(probs == m1, eidx, E), axis=-1, keepdims=True)
    mask1 = eidx == i1
    rest = jnp.where(mask1, -jnp.inf, probs)
    m2 = jnp.max(rest, axis=-1, keepdims=True)
    i2 = jnp.min(jnp.where(rest == m2, eidx, E), axis=-1, keepdims=True)
    mask2 = eidx == i2

    denom = m1 + m2
    wpe = (
        jnp.where(mask1, m1, 0.0) + jnp.where(mask2, m2, 0.0)
    ) / denom

    rmax = jnp.max(route, axis=-1, keepdims=True)
    rexp = jnp.exp(route - rmax)
    rp = rexp / jnp.sum(rexp, axis=-1, keepdims=True)
    moe_w = rp[:, 0:1]
    dense_w = rp[:, 1:2]

    cw_ref[:, :E] = moe_w * wpe
    cw_ref[:, E : E + 1] = dense_w


def _expert_body(x_ref, W1_ref, b1_ref, W2_ref, b2_ref, cw_ref, out_ref, *, BT, NE):
    e = pl.program_id(0)
    t = pl.program_id(1)
    rows = pl.ds(t * BT, BT)

    x = x_ref[rows, :]
    h = jnp.maximum(
        jnp.dot(x, W1_ref[0], preferred_element_type=jnp.float32)
        + b1_ref[0, 0, :],
        0.0,
    )
    o = (
        jnp.dot(
            h.astype(jnp.bfloat16), W2_ref[0], preferred_element_type=jnp.float32
        )
        + b2_ref[0, 0, :]
    )

    cw = cw_ref[rows, :]
    lane = jax.lax.broadcasted_iota(jnp.int32, cw.shape, 1)
    w = jnp.sum(jnp.where(lane == e, cw, 0.0), axis=-1, keepdims=True)
    contrib = w * o

    @pl.when(e == 0)
    def _():
        out_ref[rows, :] = contrib

    @pl.when(e > 0)
    def _():
        out_ref[rows, :] += contrib


def kernel(x, Wg, bg, W1, b1, W2, b2, Wd1, bd1, Wd2, bd2, Wr, br):
    B_, S_, D_ = x.shape
    T = B_ * S_
    E = W1.shape[0]
    FF = W1.shape[2]
    NE = E + 1
    xf = x.reshape(T, D_)

    Wgr = jnp.concatenate([Wg, Wr], axis=1)
    bgr = jnp.concatenate([bg, br], axis=0)

    cw = pl.pallas_call(
        functools.partial(_routing_body, E=E),
        out_shape=jax.ShapeDtypeStruct((T, NE), jnp.float32),
    )(xf, Wgr, bgr[None, :])

    W1s = jnp.concatenate([W1, Wd1[None]], axis=0).astype(jnp.bfloat16)
    b1s = jnp.concatenate([b1, bd1[None]], axis=0).reshape(NE, 1, FF)
    W2s = jnp.concatenate([W2, Wd2[None]], axis=0).astype(jnp.bfloat16)
    b2s = jnp.concatenate([b2, bd2[None]], axis=0).reshape(NE, 1, D_)
    xb = xf.astype(jnp.bfloat16)

    BT = 256
    TB = T // BT

    out = pl.pallas_call(
        functools.partial(_expert_body, BT=BT, NE=NE),
        grid=(NE, TB),
        in_specs=[
            pl.BlockSpec((T, D_), lambda e, t: (0, 0)),
            pl.BlockSpec((1, D_, FF), lambda e, t: (e, 0, 0)),
            pl.BlockSpec((1, 1, FF), lambda e, t: (e, 0, 0)),
            pl.BlockSpec((1, FF, D_), lambda e, t: (e, 0, 0)),
            pl.BlockSpec((1, 1, D_), lambda e, t: (e, 0, 0)),
            pl.BlockSpec((T, NE), lambda e, t: (0, 0)),
        ],
        out_specs=pl.BlockSpec((T, D_), lambda e, t: (0, 0)),
        out_shape=jax.ShapeDtypeStruct((T, D_), jnp.float32),
    )(xb, W1s, b1s, W2s, b2s, cw)

    return out.reshape(B_, S_, D_)


# speedup vs baseline: 1.1202x; 1.1160x over previous
import functools

import jax
import jax.numpy as jnp
from jax.experimental import pallas as pl
from jax.experimental.pallas import tpu as pltpu


def _routing_body(x_ref, Wgr_ref, bgr_ref, cw_ref, *, E):
    xf = x_ref[...]
    logits = (
        jnp.dot(xf, Wgr_ref[...], preferred_element_type=jnp.float32)
        + bgr_ref[...]
    )
    gate = logits[:, :E]
    route = logits[:, E : E + 2]

    gmax = jnp.max(gate, axis=-1, keepdims=True)
    gexp = jnp.exp(gate - gmax)
    probs = gexp / jnp.sum(gexp, axis=-1, keepdims=True)

    eidx = jax.lax.broadcasted_iota(jnp.int32, probs.shape, 1)
    m1 = jnp.max(probs, axis=-1, keepdims=True)
    i1 = jnp.min(jnp.where(probs == m1, eidx, E), axis=-1, keepdims=True)
    mask1 = eidx == i1
    rest = jnp.where(mask1, -jnp.inf, probs)
    m2 = jnp.max(rest, axis=-1, keepdims=True)
    i2 = jnp.min(jnp.where(rest == m2, eidx, E), axis=-1, keepdims=True)
    mask2 = eidx == i2

    denom = m1 + m2
    wpe = (
        jnp.where(mask1, m1, 0.0) + jnp.where(mask2, m2, 0.0)
    ) / denom

    rmax = jnp.max(route, axis=-1, keepdims=True)
    rexp = jnp.exp(route - rmax)
    rp = rexp / jnp.sum(rexp, axis=-1, keepdims=True)

    cw_ref[:, :E] = rp[:, 0:1] * wpe
    cw_ref[:, E : E + 1] = rp[:, 1:2]


def _expert_body(
    x_ref, W1_ref, W2_ref, Wd1_ref, Wd2_ref, b1_ref, b2_ref, cw_ref,
    out_ref, w1b_ref, w2b_ref, *, BT, E,
):
    e = pl.program_id(0)
    f = pl.program_id(1)
    t = pl.program_id(2)
    rows = pl.ds(t * BT, BT)

    @pl.when(t == 0)
    def _():
        @pl.when(e < E)
        def _():
            w1b_ref[...] = W1_ref[0].astype(jnp.bfloat16)
            w2b_ref[...] = W2_ref[0].astype(jnp.bfloat16)

        @pl.when(e == E)
        def _():
            w1b_ref[...] = Wd1_ref[...]
            w2b_ref[...] = Wd2_ref[...]

    x = x_ref[rows, :]
    h = jnp.maximum(
        jnp.dot(x, w1b_ref[...], preferred_element_type=jnp.float32)
        + b1_ref[0, 0, :],
        0.0,
    )
    o = jnp.dot(
        h.astype(jnp.bfloat16), w2b_ref[...], preferred_element_type=jnp.float32
    )
    o += jnp.where(f == 0, 1.0, 0.0) * b2_ref[0, 0, :]

    cw = cw_ref[rows, :]
    lane = jax.lax.broadcasted_iota(jnp.int32, cw.shape, 1)
    w = jnp.sum(jnp.where(lane == e, cw, 0.0), axis=-1, keepdims=True)
    contrib = w * o

    first = jnp.logical_and(e == 0, f == 0)

    @pl.when(first)
    def _():
        out_ref[rows, :] = contrib

    @pl.when(jnp.logical_not(first))
    def _():
        out_ref[rows, :] += contrib


def kernel(x, Wg, bg, W1, b1, W2, b2, Wd1, bd1, Wd2, bd2, Wr, br):
    B_, S_, D_ = x.shape
    T = B_ * S_
    E = W1.shape[0]
    FF = W1.shape[2]
    NE = E + 1
    FH = FF // 2
    xf = x.reshape(T, D_)

    Wgr = jnp.concatenate([Wg, Wr], axis=1)
    bgr = jnp.concatenate([bg, br], axis=0)

    cw = pl.pallas_call(
        functools.partial(_routing_body, E=E),
        out_shape=jax.ShapeDtypeStruct((T, NE), jnp.float32),
    )(xf, Wgr, bgr[None, :])

    b1s = jnp.concatenate([b1, bd1[None]], axis=0).reshape(NE, 1, FF)
    b2s = jnp.concatenate([b2, bd2[None]], axis=0).reshape(NE, 1, D_)
    Wd1b = Wd1.astype(jnp.bfloat16)
    Wd2b = Wd2.astype(jnp.bfloat16)
    xb = xf.astype(jnp.bfloat16)

    BT = 256
    TB = T // BT

    out = pl.pallas_call(
        functools.partial(_expert_body, BT=BT, E=E),
        grid=(NE, 2, TB),
        in_specs=[
            pl.BlockSpec((T, D_), lambda e, f, t: (0, 0)),
            pl.BlockSpec(
                (1, D_, FH), lambda e, f, t: (jnp.minimum(e, E - 1), 0, f)
            ),
            pl.BlockSpec(
                (1, FH, D_), lambda e, f, t: (jnp.minimum(e, E - 1), f, 0)
            ),
            pl.BlockSpec((D_, FH), lambda e, f, t: (0, jnp.where(e == E, f, 0))),
            pl.BlockSpec((FH, D_), lambda e, f, t: (jnp.where(e == E, f, 0), 0)),
            pl.BlockSpec((1, 1, FH), lambda e, f, t: (e, 0, f)),
            pl.BlockSpec((1, 1, D_), lambda e, f, t: (e, 0, 0)),
            pl.BlockSpec((T, NE), lambda e, f, t: (0, 0)),
        ],
        out_specs=pl.BlockSpec((T, D_), lambda e, f, t: (0, 0)),
        out_shape=jax.ShapeDtypeStruct((T, D_), jnp.float32),
        scratch_shapes=[
            pltpu.VMEM((D_, FH), jnp.bfloat16),
            pltpu.VMEM((FH, D_), jnp.bfloat16),
        ],
    )(xb, W1, W2, Wd1b, Wd2b, b1s, b2s, cw)

    return out.reshape(B_, S_, D_)
